# baseline (speedup 1.0000x reference)
import functools

import jax
import jax.numpy as jnp
from jax import lax
from jax.experimental import pallas as pl
from jax.experimental.pallas import tpu as pltpu
from jax.experimental.pallas import tpu_sc as plsc

_K = 8192
_D = 32
_N = 8192
_BLK = 256
_NBLK = _N // _BLK
_COMMIT = 0.25

_SC_NC = 2
_SC_NS = 16
_NW = _SC_NC * _SC_NS
_BPW = _N // _NW
_CHUNK = 128
_NCHUNK = _BPW // _CHUNK
_HL = 128
_NROWS = _N // _CHUNK


def _vq_main_body(x_ref, w_ref,
                  soft_ref, idx_ref, wn_ref, loss_ref,
                  acc_ref, wm2_ref):
    i = pl.program_id(0)

    @pl.when(i == 0)
    def _init():
        w = w_ref[...]
        wn = w / jnp.maximum(jnp.sqrt(jnp.sum(w * w, axis=1, keepdims=True)),
                             1e-12)
        wn_ref[...] = jnp.concatenate(
            [wn, jnp.zeros((_K, 128 - _D), jnp.float32)], axis=1)
        wm2_ref[...] = wn * -2.0
        acc_ref[0, 0] = 0.0

    x = x_ref[...]
    fn = x / jnp.maximum(jnp.sqrt(jnp.sum(x * x, axis=1, keepdims=True)), 1e-12)

    t = jax.lax.dot_general(fn, wm2_ref[...], (((1,), (1,)), ((), ())),
                            preferred_element_type=jnp.float32)
    d = 2.0 + t

    dmin = jnp.min(d, axis=1, keepdims=True)
    e = jnp.exp(dmin - d)
    s = jnp.sum(e, axis=1, keepdims=True)
    soft_ref[...] = e / s

    idx = jnp.argmin(d, axis=1).astype(jnp.int32)
    idx_ref[...] = idx[None, :]

    acc_ref[0, 0] += jnp.sum(dmin)

    @pl.when(i == _NBLK - 1)
    def _finish():
        loss_ref[0, 0] = _COMMIT * acc_ref[0, 0] / (_N * _D)


def _sc_body(wn_hbm, idx_hbm, zeros_hbm, ones_hbm,
             quant_hbm, hist_hbm,
             idx_v, idxall_v, rows_v, ones_v, hist_sh, sem):
    c = lax.axis_index("c")
    s = lax.axis_index("s")
    wid = s * _SC_NC + c

    pltpu.sync_copy(idx_hbm.at[pl.ds(wid * _NCHUNK, _NCHUNK)], idx_v)
    for j in range(_NCHUNK):
        pltpu.async_copy(wn_hbm.at[idx_v.at[j]],
                         rows_v.at[pl.ds(j * _CHUNK, _CHUNK)], sem).wait()
    pltpu.sync_copy(rows_v, quant_hbm.at[pl.ds(wid * _BPW, _BPW)])

    @pl.when(s == 0)
    def _hist():
        pltpu.sync_copy(ones_hbm, ones_v)
        for k in range(_K // 512):
            pltpu.sync_copy(zeros_hbm, hist_sh.at[pl.ds(k * 512, 512)])
        pltpu.sync_copy(idx_hbm.at[pl.ds(c * (_NROWS // 2), _NROWS // 2)],
                        idxall_v)
        for j in range(_NROWS // 2):
            pltpu.sync_copy(ones_v, hist_sh.at[idxall_v.at[j]], add=True)
        pltpu.sync_copy(hist_sh, hist_hbm.at[pl.ds(c * _K, _K)])


def _stats_body(h_ref, perp_ref, active_ref, usage_ref):
    h = h_ref[0] + h_ref[1]
    lane = jax.lax.broadcasted_iota(jnp.int32, (128, 128 // _HL), 0)
    binc = jax.lax.broadcasted_iota(jnp.int32, (128, 128 // _HL), 1)
    group = jnp.where(lane // _HL == binc, 1.0 / _HL, 0.0)
    counts = jax.lax.dot_general(h, group, (((1,), (0,)), ((), ())),
                                 preferred_element_type=jnp.float32)
    avg = counts / _N
    perp_ref[0, 0] = jnp.exp(-jnp.sum(avg * jnp.log(avg + 1e-10)))
    active = jnp.sum((counts > 0.0).astype(jnp.int32))
    active_ref[0, 0] = active
    usage_ref[0, 0] = active.astype(jnp.float32) / _K * 100.0


@jax.jit
def kernel(inputs, weight):
    flat = inputs.reshape(_N, _D)

    soft, idx, wn, loss = pl.pallas_call(
        _vq_main_body,
        grid=(_NBLK,),
        in_specs=[
            pl.BlockSpec((_BLK, _D), lambda i: (i, 0)),
            pl.BlockSpec((_K, _D), lambda i: (0, 0)),
        ],
        out_specs=(
            pl.BlockSpec((_BLK, _K), lambda i: (i, 0)),
            pl.BlockSpec((1, _BLK), lambda i: (0, i)),
            pl.BlockSpec((_K, 128), lambda i: (0, 0)),
            pl.BlockSpec(memory_space=pltpu.SMEM),
        ),
        out_shape=(
            jax.ShapeDtypeStruct((_N, _K), jnp.float32),
            jax.ShapeDtypeStruct((1, _N), jnp.int32),
            jax.ShapeDtypeStruct((_K, 128), jnp.float32),
            jax.ShapeDtypeStruct((1, 1), jnp.float32),
        ),
        scratch_shapes=[
            pltpu.SMEM((1, 1), jnp.float32),
            pltpu.VMEM((_K, _D), jnp.float32),
        ],
        compiler_params=pltpu.CompilerParams(
            dimension_semantics=("arbitrary",),
        ),
    )(flat, weight)

    sc_gather_hist = functools.partial(
        pl.kernel,
        mesh=plsc.VectorSubcoreMesh(core_axis_name="c", subcore_axis_name="s"),
        out_type=(
            jax.ShapeDtypeStruct((_N, 128), jnp.float32),
            jax.ShapeDtypeStruct((_SC_NC * _K, _HL), jnp.float32),
        ),
        scratch_types=[
            pltpu.VMEM((_NCHUNK, _CHUNK), jnp.int32),
            pltpu.VMEM((_NROWS // 2, _CHUNK), jnp.int32),
            pltpu.VMEM((_BPW, 128), jnp.float32),
            pltpu.VMEM((_CHUNK, _HL), jnp.float32),
            pltpu.VMEM_SHARED((_K, _HL), jnp.float32),
            pltpu.SemaphoreType.DMA,
        ],
    )(_sc_body)
    quant, hist = sc_gather_hist(
        wn, idx.reshape(_N // _CHUNK, _CHUNK),
        jnp.zeros((512, _HL), jnp.float32), jnp.ones((_CHUNK, _HL), jnp.float32))

    perp, active, usage = pl.pallas_call(
        _stats_body,
        out_specs=(
            pl.BlockSpec(memory_space=pltpu.SMEM),
            pl.BlockSpec(memory_space=pltpu.SMEM),
            pl.BlockSpec(memory_space=pltpu.SMEM),
        ),
        out_shape=(
            jax.ShapeDtypeStruct((1, 1), jnp.float32),
            jax.ShapeDtypeStruct((1, 1), jnp.int32),
            jax.ShapeDtypeStruct((1, 1), jnp.float32),
        ),
    )(hist.reshape(_SC_NC, _K * _HL // 128, 128))

    return (
        loss.reshape(()),
        quant[:, :_D].reshape(inputs.shape),
        soft,
        perp.reshape(()),
        idx.reshape(_N, 1),
        active.reshape(()),
        usage.reshape(()),
    )
